# baseline (speedup 1.0000x reference)
import functools

import jax
import jax.numpy as jnp
from jax import lax
from jax.experimental import pallas as pl
from jax.experimental.pallas import tpu as pltpu
from jax.experimental.pallas import tpu_sc as plsc

H = 512
HALF = 384
CS = 128
NPART = 6
FULL = 2 * HALF

_info = plsc.get_sparse_core_info()
_NC, _NS, _L = _info.num_cores, _info.num_subcores, _info.num_lanes


def _make_lookup(n_tok: int):
  NG = _NC * 2
  SLAB = n_tok // NG
  T = 160
  ROUNDS = SLAB // T
  assert n_tok == NG * SLAB and SLAB % T == 0 and ROUNDS % 2 == 0
  mesh = plsc.VectorSubcoreMesh(core_axis_name="c", subcore_axis_name="s")

  @functools.partial(
      pl.kernel,
      mesh=mesh,
      compiler_params=pltpu.CompilerParams(needs_layout_passes=False),
      out_type=jax.ShapeDtypeStruct((n_tok, FULL), jnp.float32),
      scratch_types=[
          pltpu.VMEM((H, CS), jnp.float32),
          pltpu.VMEM((SLAB,), jnp.int32),
          pltpu.VMEM((T, CS), jnp.float32),
          pltpu.VMEM((T, CS), jnp.float32),
          pltpu.SemaphoreType.DMA,
          pltpu.SemaphoreType.DMA,
      ],
  )
  def k(table6_hbm, tok2_hbm, out_hbm, tbl_v, idx_v, stage0, stage1, w0, w1):
    stage = (stage0, stage1)
    cid = lax.axis_index("c")
    sid = lax.axis_index("s")
    q = sid // 8
    p = sid % 8
    active = p < NPART
    pidx = p // 3
    slab = (cid * 2 + q) * SLAB
    wsem = (w0, w1)

    iota = lax.iota(jnp.int32, _L)
    zeros = iota - iota

    def start_write(r, b):
      return pltpu.async_copy(
          stage[b],
          out_hbm.at[pl.ds(slab + r * T, T), pl.ds(p * CS, CS)], wsem[b])

    def wait_write(b):
      pltpu.make_async_copy(
          stage[b],
          out_hbm.at[pl.ds(slab, T), pl.ds(p * CS, CS)], wsem[b]).wait()

    def round_body(r, b, drain):
      off = r * T
      if drain:
        wait_write(b)
      @plsc.parallel_loop(0, T, step=_L, unroll=2)
      def _(t):
        idx16 = idx_v[pl.ds(off + t, _L)]
        for i in range(_L):
          trow = tbl_v.at[idx16[i]]
          srow = stage[b].at[t + i]
          for j in range(CS // _L):
            srow[pl.ds(j * _L, _L)] = trow[pl.ds(j * _L, _L)]

      start_write(r, b)

    @pl.when(active)
    def _():
      pltpu.sync_copy(table6_hbm.at[p], tbl_v)
      pltpu.sync_copy(tok2_hbm.at[pidx, pl.ds(slab, SLAB)], idx_v)

      round_body(0, 0, drain=False)
      round_body(1, 1, drain=False)

      @pl.loop(2, ROUNDS, step=2)
      def _(o):
        round_body(o, 0, drain=True)
        round_body(o + 1, 1, drain=True)

      wait_write(0)
      wait_write(1)

  return k


_lookup = _make_lookup(1024 * 50)


def kernel(token, row_embed, col_embed):
  B, L, _ = token.shape
  n_tok = B * L
  stacked = jnp.stack([row_embed, col_embed])
  table6 = stacked.reshape(2, H, 3, CS).transpose(0, 2, 1, 3).reshape(
      NPART, H, CS)
  tok2 = token.astype(jnp.int32).reshape(n_tok, 2).T
  out = _lookup(table6, tok2)
  return out.reshape(B, L, FULL)


# speedup vs baseline: 1.0289x; 1.0289x over previous
import functools

import jax
import jax.numpy as jnp
from jax import lax
from jax.experimental import pallas as pl
from jax.experimental.pallas import tpu as pltpu
from jax.experimental.pallas import tpu_sc as plsc

H = 512
HALF = 384

_info = plsc.get_sparse_core_info()
_NC, _NS, _L = _info.num_cores, _info.num_subcores, _info.num_lanes
_NW = _NC * _NS
_R = 40


def _make_gather(n_rows: int):
  R = _R
  NBG = 4
  NBW = 2
  G = n_rows // (_NW * R)
  MAIN = ((G - 2 * NBG) // NBG) * NBG
  assert n_rows == G * _NW * R and G >= 2 * NBG and MAIN > 0
  mesh = plsc.VectorSubcoreMesh(core_axis_name="c", subcore_axis_name="s")

  @functools.partial(
      pl.kernel,
      mesh=mesh,
      out_type=jax.ShapeDtypeStruct((G, _NC, _NS, R, HALF), jnp.float32),
      scratch_types=[
          pltpu.VMEM((G, R), jnp.int32),
          pltpu.VMEM((NBG, R, HALF), jnp.float32),
          pltpu.VMEM_SHARED((NBW, _NS, R, HALF), jnp.float32),
          pltpu.SemaphoreType.DMA,
          pltpu.SemaphoreType.DMA,
          pltpu.SemaphoreType.DMA,
          pltpu.SemaphoreType.DMA,
          pltpu.SemaphoreType.DMA,
          pltpu.SemaphoreType.DMA,
      ],
  )
  def k(table_hbm, idx_hbm, out_hbm, idx_v, rows_v, shared,
        g0, g1, g2, g3, w0, w1):
    cid = lax.axis_index("c")
    sid = lax.axis_index("s")
    gsem = (g0, g1, g2, g3)
    wsem = (w0, w1)

    pltpu.sync_copy(idx_hbm.at[:, cid, sid], idx_v)
    offs = (lax.iota(jnp.int32, _L) & 1) * H

    @pl.loop(0, G)
    def _(g):
      @pl.loop(0, R, step=_L)
      def _(i):
        sl = pl.ds(i, _L)
        idx_v[g, sl] = idx_v[g, sl] + offs

    def start_gather(g, bg):
      return pltpu.async_copy(
          table_hbm.at[idx_v.at[g]], rows_v.at[bg], gsem[bg])

    def wait_gather(bg):
      pltpu.make_async_copy(
          table_hbm.at[idx_v.at[0]], rows_v.at[bg], gsem[bg]).wait()

    def start_write(g, bw):
      return pltpu.async_copy(shared.at[bw], out_hbm.at[g, cid], wsem[bw])

    def wait_write(bw):
      pltpu.make_async_copy(
          shared.at[bw], out_hbm.at[0, cid], wsem[bw]).wait()

    def round_body(g, bg, bw, drain, prefetch):
      wait_gather(bg)
      if drain:
        @pl.when(sid == 0)
        def _():
          wait_write(bw)
      plsc.subcore_barrier()
      pltpu.sync_copy(rows_v.at[bg], shared.at[bw, sid])
      if prefetch:
        start_gather(g + NBG, bg)
      plsc.subcore_barrier()

      @pl.when(sid == 0)
      def _():
        start_write(g, bw)

    for b in range(NBG):
      start_gather(b, b)
    for g in range(NBG):
      round_body(g, g % NBG, g % NBW, drain=(g >= NBW), prefetch=True)

    @pl.loop(NBG, NBG + MAIN, step=NBG)
    def _(o):
      for b in range(NBG):
        round_body(o + b, b, b % NBW, drain=True, prefetch=True)

    for g in range(NBG + MAIN, G):
      round_body(g, g % NBG, g % NBW, drain=True, prefetch=(g + NBG < G))

    @pl.when(sid == 0)
    def _():
      for b in range(NBW):
        wait_write(b)

    plsc.subcore_barrier()

  return k


_gather = _make_gather(2 * 1024 * 50)
_G = 2 * 1024 * 50 // (_NW * _R)


def kernel(token, row_embed, col_embed):
  B, L, _ = token.shape
  table = jnp.concatenate([row_embed, col_embed], axis=0)
  idx = token.astype(jnp.int32).reshape(_G, _NC, _NS, _R)
  out = _gather(table, idx)
  return out.reshape(B, L, 2 * HALF)
